# baseline (speedup 1.0000x reference)
import functools

import jax
import jax.numpy as jnp
from jax import lax
from jax.experimental import pallas as pl
from jax.experimental.pallas import tpu as pltpu
from jax.experimental.pallas import tpu_sc as plsc

NUM = 1_000_000
DIM = 64
BATCH = 16384
FEAT = 26

NC = 2
NS = 16
NW = NC * NS

BPW = BATCH // NW
GROUP = 128
NG = FEAT * (BPW // GROUP)
RPW = FEAT * BPW
L = 16
NBUF = 4

SPLIT = 512_000
CB = 2048
TGRID = SPLIT // CB


def _tc_pack(w_t):
    def body(in1, in2, out):
        out[...] = jnp.concatenate(
            [jnp.transpose(in1[...]), jnp.transpose(in2[...])], axis=1)

    return pl.pallas_call(
        body,
        grid=(TGRID,),
        in_specs=[
            pl.BlockSpec((DIM, CB), lambda i: (0, i)),
            pl.BlockSpec(
                (DIM, CB),
                lambda i: (0, jnp.minimum(i + TGRID, NUM // CB))),
        ],
        out_specs=pl.BlockSpec((CB, 2 * DIM), lambda i: (i, 0)),
        out_shape=jax.ShapeDtypeStruct((SPLIT, 2 * DIM), jnp.float32),
    )(w_t, w_t)


def _sc_gather(idx_flat, table_v):
    mesh = plsc.VectorSubcoreMesh(core_axis_name="c", subcore_axis_name="s")

    @functools.partial(
        pl.kernel,
        mesh=mesh,
        out_type=jax.ShapeDtypeStruct((BATCH, FEAT, DIM), jnp.float32),
        scratch_types=[
            pltpu.VMEM((RPW,), jnp.int32),
            *[pltpu.VMEM((GROUP, 2 * DIM), jnp.float32) for _ in range(NBUF)],
            *[pltpu.VMEM((GROUP, DIM), jnp.float32) for _ in range(2)],
            *[pltpu.VMEM((GROUP,), jnp.int32) for _ in range(NBUF)],
            *[pltpu.SemaphoreType.DMA for _ in range(NBUF + 2)],
        ],
    )
    def k(idx_hbm, w_hbm, out_hbm, idx_v, *bufs):
        gbufs = bufs[:NBUF]
        cbufs = bufs[NBUF:NBUF + 2]
        rbufs = bufs[NBUF + 2:2 * NBUF + 2]
        gsems = bufs[2 * NBUF + 2:3 * NBUF + 2]
        wsems = bufs[3 * NBUF + 2:3 * NBUF + 4]
        wid = lax.axis_index("s") * NC + lax.axis_index("c")
        b0 = wid * BPW

        for f in range(FEAT):
            pltpu.sync_copy(idx_hbm.at[pl.ds(f * BATCH + b0, BPW)],
                            idx_v.at[pl.ds(f * BPW, BPW)])

        def fire_gather(g, p):
            rb = rbufs[p]
            for s in range(GROUP // L):
                v = idx_v[pl.ds(g * GROUP + s * L, L)]
                rb[pl.ds(s * L, L)] = v - jnp.where(v >= SPLIT, SPLIT, 0)
            pltpu.async_copy(w_hbm.at[rb], gbufs[p], gsems[p])

        def wait_gather(g, p):
            pltpu.make_async_copy(w_hbm.at[rbufs[p]], gbufs[p],
                                  gsems[p]).wait()

        def out_slice(g):
            f = lax.shift_right_logical(g, 2)
            j = lax.bitwise_and(g, 3)
            return out_hbm.at[pl.ds(b0 + j * GROUP, GROUP), f]

        def fire_write(g, q):
            pltpu.async_copy(cbufs[q], out_slice(g), wsems[q])

        def wait_write(g, q):
            pltpu.make_async_copy(cbufs[q], out_slice(g), wsems[q]).wait()

        def compact(g, p, q):
            gb = gbufs[p]
            cb = cbufs[q]

            def stripe(s, carry):
                base = g * GROUP + s * L
                hv = jnp.where(idx_v[pl.ds(base, L)] >= SPLIT, DIM, 0)
                for rr in range(L):
                    r = s * L + rr
                    h = hv[rr]
                    for kk in range(DIM // L):
                        cb[r, pl.ds(kk * L, L)] = gb[r, pl.ds(h + kk * L, L)]
                return carry

            lax.fori_loop(0, GROUP // L, stripe, 0)

        for p in range(NBUF - 1):
            fire_gather(p, p)

        def quad(t, carry):
            for p in range(NBUF):
                g = NBUF * t + p

                @pl.when(g + NBUF - 1 < NG)
                def _():
                    fire_gather(g + NBUF - 1, (p + NBUF - 1) % NBUF)

                wait_gather(g, p)

                @pl.when(g >= 2)
                def _():
                    wait_write(g - 2, p % 2)

                compact(g, p, p % 2)
                fire_write(g, p % 2)
            return carry

        lax.fori_loop(0, NG // NBUF, quad, 0)
        wait_write(NG - 2, 0)
        wait_write(NG - 1, 1)

    return k(idx_flat, table_v)


def kernel(input, W):
    idx_flat = jnp.transpose(input.astype(jnp.int32)).reshape(-1)
    table_v = _tc_pack(jnp.transpose(W))
    return _sc_gather(idx_flat, table_v)


# speedup vs baseline: 1.0966x; 1.0966x over previous
import functools

import jax
import jax.numpy as jnp
from jax import lax
from jax.experimental import pallas as pl
from jax.experimental.pallas import tpu as pltpu
from jax.experimental.pallas import tpu_sc as plsc

NUM = 1_000_000
DIM = 64
BATCH = 16384
FEAT = 26

NC = 2
NS = 16
NW = NC * NS

BPW = BATCH // NW
GROUP = 128
NG = FEAT * (BPW // GROUP)
RPW = FEAT * BPW
L = 16
NBUF = 4

SPLIT = 512_000
CB = 4096
TGRID = SPLIT // CB


def _tc_pack(w_t):
    def body(in1, in2, out):
        out[...] = jnp.concatenate(
            [jnp.transpose(in1[...]), jnp.transpose(in2[...])], axis=1)

    return pl.pallas_call(
        body,
        grid=(TGRID,),
        in_specs=[
            pl.BlockSpec((DIM, CB), lambda i: (0, i)),
            pl.BlockSpec(
                (DIM, CB),
                lambda i: (0, jnp.minimum(i + TGRID, NUM // CB))),
        ],
        out_specs=pl.BlockSpec((CB, 2 * DIM), lambda i: (i, 0)),
        out_shape=jax.ShapeDtypeStruct((SPLIT, 2 * DIM), jnp.float32),
    )(w_t, w_t)


def _sc_gather(idx_flat, table_v):
    mesh = plsc.VectorSubcoreMesh(core_axis_name="c", subcore_axis_name="s")

    @functools.partial(
        pl.kernel,
        mesh=mesh,
        out_type=jax.ShapeDtypeStruct((BATCH, FEAT, DIM), jnp.float32),
        scratch_types=[
            pltpu.VMEM((RPW,), jnp.int32),
            *[pltpu.VMEM((GROUP, 2 * DIM), jnp.float32) for _ in range(NBUF)],
            *[pltpu.VMEM((GROUP, DIM), jnp.float32) for _ in range(2)],
            *[pltpu.VMEM((GROUP,), jnp.int32) for _ in range(NBUF)],
            *[pltpu.SemaphoreType.DMA for _ in range(NBUF + 2)],
        ],
    )
    def k(idx_hbm, w_hbm, out_hbm, idx_v, *bufs):
        gbufs = bufs[:NBUF]
        cbufs = bufs[NBUF:NBUF + 2]
        rbufs = bufs[NBUF + 2:2 * NBUF + 2]
        gsems = bufs[2 * NBUF + 2:3 * NBUF + 2]
        wsems = bufs[3 * NBUF + 2:3 * NBUF + 4]
        wid = lax.axis_index("s") * NC + lax.axis_index("c")
        b0 = wid * BPW

        for f in range(FEAT):
            pltpu.sync_copy(idx_hbm.at[pl.ds(f * BATCH + b0, BPW)],
                            idx_v.at[pl.ds(f * BPW, BPW)])

        def fire_gather(g, p):
            rb = rbufs[p]
            for s in range(GROUP // L):
                v = idx_v[pl.ds(g * GROUP + s * L, L)]
                rb[pl.ds(s * L, L)] = v - jnp.where(v >= SPLIT, SPLIT, 0)
            pltpu.async_copy(w_hbm.at[rb], gbufs[p], gsems[p])

        def wait_gather(g, p):
            pltpu.make_async_copy(w_hbm.at[rbufs[p]], gbufs[p],
                                  gsems[p]).wait()

        def out_slice(g):
            f = lax.shift_right_logical(g, 2)
            j = lax.bitwise_and(g, 3)
            return out_hbm.at[pl.ds(b0 + j * GROUP, GROUP), f]

        def fire_write(g, q):
            pltpu.async_copy(cbufs[q], out_slice(g), wsems[q])

        def wait_write(g, q):
            pltpu.make_async_copy(cbufs[q], out_slice(g), wsems[q]).wait()

        def compact(g, p, q):
            gb = gbufs[p]
            cb = cbufs[q]

            def stripe(s, carry):
                base = g * GROUP + s * L
                hv = jnp.where(idx_v[pl.ds(base, L)] >= SPLIT, DIM, 0)
                for rr in range(L):
                    r = s * L + rr
                    h = hv[rr]
                    for kk in range(DIM // L):
                        cb[r, pl.ds(kk * L, L)] = gb[r, pl.ds(h + kk * L, L)]
                return carry

            lax.fori_loop(0, GROUP // L, stripe, 0)

        for p in range(NBUF - 1):
            fire_gather(p, p)

        def quad(t, carry):
            for p in range(NBUF):
                g = NBUF * t + p

                @pl.when(g + NBUF - 1 < NG)
                def _():
                    fire_gather(g + NBUF - 1, (p + NBUF - 1) % NBUF)

                wait_gather(g, p)

                @pl.when(g >= 2)
                def _():
                    wait_write(g - 2, p % 2)

                compact(g, p, p % 2)
                fire_write(g, p % 2)
            return carry

        lax.fori_loop(0, NG // NBUF, quad, 0)
        wait_write(NG - 2, 0)
        wait_write(NG - 1, 1)

    return k(idx_flat, table_v)


def kernel(input, W):
    idx_flat = jnp.transpose(input.astype(jnp.int32)).reshape(-1)
    table_v = _tc_pack(jnp.transpose(W))
    return _sc_gather(idx_flat, table_v)


# speedup vs baseline: 1.1776x; 1.0739x over previous
import functools

import jax
import jax.numpy as jnp
from jax import lax
from jax.experimental import pallas as pl
from jax.experimental.pallas import tpu as pltpu
from jax.experimental.pallas import tpu_sc as plsc

NUM = 1_000_000
DIM = 64
BATCH = 16384
FEAT = 26

NC = 2
NS = 16
NW = NC * NS

BPW = BATCH // NW
GROUP = 128
NG = FEAT * (BPW // GROUP)
RPW = FEAT * BPW
L = 16
NBUF = 4

SPLIT = 512_000
CB = 12800
TGRID = SPLIT // CB


def _tc_pack(w_t):
    def body(in1, in2, out):
        out[...] = jnp.concatenate(
            [jnp.transpose(in1[...]), jnp.transpose(in2[...])], axis=1)

    return pl.pallas_call(
        body,
        grid=(TGRID,),
        in_specs=[
            pl.BlockSpec((DIM, CB), lambda i: (0, i)),
            pl.BlockSpec(
                (DIM, CB),
                lambda i: (0, jnp.minimum(i + TGRID, NUM // CB))),
        ],
        out_specs=pl.BlockSpec((CB, 2 * DIM), lambda i: (i, 0)),
        out_shape=jax.ShapeDtypeStruct((SPLIT, 2 * DIM), jnp.float32),
    )(w_t, w_t)


def _sc_gather(idx_flat, table_v):
    mesh = plsc.VectorSubcoreMesh(core_axis_name="c", subcore_axis_name="s")

    @functools.partial(
        pl.kernel,
        mesh=mesh,
        out_type=jax.ShapeDtypeStruct((BATCH, FEAT, DIM), jnp.float32),
        scratch_types=[
            pltpu.VMEM((RPW,), jnp.int32),
            *[pltpu.VMEM((GROUP, 2 * DIM), jnp.float32) for _ in range(NBUF)],
            *[pltpu.VMEM((GROUP, DIM), jnp.float32) for _ in range(2)],
            *[pltpu.VMEM((GROUP,), jnp.int32) for _ in range(NBUF)],
            *[pltpu.SemaphoreType.DMA for _ in range(NBUF + 2)],
        ],
    )
    def k(idx_hbm, w_hbm, out_hbm, idx_v, *bufs):
        gbufs = bufs[:NBUF]
        cbufs = bufs[NBUF:NBUF + 2]
        rbufs = bufs[NBUF + 2:2 * NBUF + 2]
        gsems = bufs[2 * NBUF + 2:3 * NBUF + 2]
        wsems = bufs[3 * NBUF + 2:3 * NBUF + 4]
        wid = lax.axis_index("s") * NC + lax.axis_index("c")
        b0 = wid * BPW

        for f in range(FEAT):
            pltpu.sync_copy(idx_hbm.at[pl.ds(f * BATCH + b0, BPW)],
                            idx_v.at[pl.ds(f * BPW, BPW)])

        def fire_gather(g, p):
            rb = rbufs[p]
            for s in range(GROUP // L):
                v = idx_v[pl.ds(g * GROUP + s * L, L)]
                rb[pl.ds(s * L, L)] = v - jnp.where(v >= SPLIT, SPLIT, 0)
            pltpu.async_copy(w_hbm.at[rb], gbufs[p], gsems[p])

        def wait_gather(g, p):
            pltpu.make_async_copy(w_hbm.at[rbufs[p]], gbufs[p],
                                  gsems[p]).wait()

        def out_slice(g):
            f = lax.shift_right_logical(g, 2)
            j = lax.bitwise_and(g, 3)
            return out_hbm.at[pl.ds(b0 + j * GROUP, GROUP), f]

        def fire_write(g, q):
            pltpu.async_copy(cbufs[q], out_slice(g), wsems[q])

        def wait_write(g, q):
            pltpu.make_async_copy(cbufs[q], out_slice(g), wsems[q]).wait()

        def compact(g, p, q):
            gb = gbufs[p]
            cb = cbufs[q]

            def stripe(s, carry):
                base = g * GROUP + s * L
                hv = jnp.where(idx_v[pl.ds(base, L)] >= SPLIT, DIM, 0)
                for rr in range(L):
                    r = s * L + rr
                    h = hv[rr]
                    for kk in range(DIM // L):
                        cb[r, pl.ds(kk * L, L)] = gb[r, pl.ds(h + kk * L, L)]
                return carry

            lax.fori_loop(0, GROUP // L, stripe, 0)

        for p in range(NBUF - 1):
            fire_gather(p, p)

        def quad(t, carry):
            for p in range(NBUF):
                g = NBUF * t + p

                @pl.when(g + NBUF - 1 < NG)
                def _():
                    fire_gather(g + NBUF - 1, (p + NBUF - 1) % NBUF)

                wait_gather(g, p)

                @pl.when(g >= 2)
                def _():
                    wait_write(g - 2, p % 2)

                compact(g, p, p % 2)
                fire_write(g, p % 2)
            return carry

        lax.fori_loop(0, NG // NBUF, quad, 0)
        wait_write(NG - 2, 0)
        wait_write(NG - 1, 1)

    return k(idx_flat, table_v)


def kernel(input, W):
    idx_flat = jnp.transpose(input.astype(jnp.int32)).reshape(-1)
    table_v = _tc_pack(jnp.transpose(W))
    return _sc_gather(idx_flat, table_v)
